# Initial kernel scaffold; baseline (speedup 1.0000x reference)
import jax
import jax.numpy as jnp
from jax.experimental import pallas as pl


def kernel(feat0, feat1, W_p0, b_p0, g_p0, be_p0, W_p1, b_p1, g_p1, be_p1, W_in, b_in, W_out, b_out, W_e1, b_e1, W_e2, b_e2):
    raise NotImplementedError("write your pallas kernel here")


# speedup vs baseline: 8.1613x; 8.1613x over previous
import jax
import jax.numpy as jnp
from jax.experimental import pallas as pl

_B = 4096
_D = 256
_H = 128
_NH = 4
_HD = 32
_K = 8

_TRA = 512
_CCH = 128
_TCB = 256

_SCALE = 1.0 / (_HD ** 0.5)
_HI = jax.lax.Precision.HIGHEST


def _bfr(x):
    return x.astype(jnp.bfloat16).astype(jnp.float32)


def _dot(a, b, prec=None):
    return jax.lax.dot_general(a, b, (((1,), (1,)), ((), ())),
                               preferred_element_type=jnp.float32,
                               precision=prec)


def _frontend_body(f0, f1, wp0, bp0, gp0, bep0, wp1, bp1, gp1, bep1,
                   win, bin_, wout, bout, we1, be1, we2, be2,
                   fused_ref, ew_ref):
    inv_h = 1.0 / _H

    def proj_ln(x, w, b, g, be):
        h = jnp.maximum(_dot(x[...], w[...]) + b[...], 0.0)
        m = jnp.sum(h, axis=1, keepdims=True) * inv_h
        d = h - m
        v = jnp.sum(d * d, axis=1, keepdims=True) * inv_h
        return _bfr(d / jnp.sqrt(v + 1e-5) * g[...] + be[...])

    p0 = proj_ln(f0, wp0, bp0, gp0, bep0)
    p1 = proj_ln(f1, wp1, bp1, gp1, bep1)

    qkv0 = _bfr(_dot(p0, win[...]) + bin_[...])
    qkv1 = _bfr(_dot(p1, win[...]) + bin_[...])
    q = (qkv0[:, 0:_H], qkv1[:, 0:_H])
    k = (qkv0[:, _H:2 * _H], qkv1[:, _H:2 * _H])
    v = (qkv0[:, 2 * _H:3 * _H], qkv1[:, 2 * _H:3 * _H])

    di = jax.lax.broadcasted_iota(jnp.int32, (_H, _NH), 0)
    hi = jax.lax.broadcasted_iota(jnp.int32, (_H, _NH), 1)
    hmat = (di // _HD == hi).astype(jnp.float32)

    lane = jax.lax.broadcasted_iota(jnp.int32, (_TRA, _H), 1)
    ri = jax.lax.broadcasted_iota(jnp.int32, (_CCH, _CCH), 0)
    ci = jax.lax.broadcasted_iota(jnp.int32, (_CCH, _CCH), 1)
    dmask = ri == ci

    def scores(qa, kb):
        cols = []
        for h in range(_NH):
            kbh = jnp.where((lane // _HD) == h, kb, 0.0)
            chunks = []
            for c in range(_TRA // _CCH):
                lo, hi_ = c * _CCH, (c + 1) * _CCH
                s = _dot(qa[lo:hi_, :], kbh[lo:hi_, :])
                chunks.append(jnp.sum(jnp.where(dmask, s, 0.0),
                                      axis=1, keepdims=True))
            cols.append(jnp.concatenate(chunks, axis=0))
        return jnp.concatenate(cols, axis=1)

    def attn_pair(a):
        sa0 = scores(q[a], k[0]) * _SCALE
        sa1 = scores(q[a], k[1]) * _SCALE
        m = jnp.maximum(sa0, sa1)
        e0 = jnp.exp(sa0 - m)
        e1 = jnp.exp(sa1 - m)
        den = e0 + e1
        return _bfr(e0 / den), _bfr(e1 / den)

    def hexp(w):
        return _dot(w, hmat)

    w00, w01 = attn_pair(0)
    w10, w11 = attn_pair(1)
    o0 = _bfr(hexp(w00) * v[0] + hexp(w01) * v[1])
    o1 = _bfr(hexp(w10) * v[0] + hexp(w11) * v[1])
    att0 = _dot(o0, wout[...]) + bout[...]
    att1 = _dot(o1, wout[...]) + bout[...]
    fused = (att0 + att1) * 0.5
    fused_ref[...] = fused

    hdn = jnp.maximum(_dot(fused, we1[...]) + be1[...], 0.0)
    lg = _dot(hdn, we2[...]) + be2[...]
    ew = 1.0 / (1.0 + jnp.exp(-lg))
    ew_ref[...] = jnp.maximum(ew, 1e-8)


def _incidence_body(fc, fall, out_ref):
    c = pl.program_id(0)
    logits = _dot(fc[...], fall[...])
    m0 = jnp.max(logits, axis=1, keepdims=True)
    e = jnp.exp(logits - m0)
    ssum = jnp.sum(e, axis=1, keepdims=True)
    x = e / ssum

    iota_f = jax.lax.broadcasted_iota(jnp.int32, (_TCB, _B), 1).astype(jnp.float32)
    vals = []
    idxs = []
    for _ in range(_K):
        m = jnp.max(x, axis=1, keepdims=True)
        ik = jnp.min(jnp.where(x == m, iota_f, 3.0e9), axis=1, keepdims=True)
        x = jnp.where(iota_f == ik, -1.0, x)
        vals.append(m)
        idxs.append(ik)
    V = jnp.concatenate(vals, axis=1)
    I = jnp.concatenate(idxs, axis=1)

    e1 = jax.lax.broadcasted_iota(jnp.int32, (_K, _K), 0)
    e2 = jax.lax.broadcasted_iota(jnp.int32, (_K, _K), 1)
    eye = (e1 == e2).astype(jnp.float32)
    VT = _dot(eye, V, _HI)
    IT = _dot(eye, I, _HI)

    ITi = IT.astype(jnp.int32)
    riota = jax.lax.broadcasted_iota(jnp.int32, (_B, _TCB), 0)
    cglob = jax.lax.broadcasted_iota(jnp.int32, (_B, _TCB), 1) + c * _TCB
    acc = jnp.where(riota == cglob, 1.0, 0.0)
    for kk in range(_K):
        acc = jnp.where(riota == ITi[kk:kk + 1, :], VT[kk:kk + 1, :], acc)
    out_ref[...] = acc


def kernel(feat0, feat1, W_p0, b_p0, g_p0, be_p0, W_p1, b_p1, g_p1, be_p1,
           W_in, b_in, W_out, b_out, W_e1, b_e1, W_e2, b_e2):
    r1 = lambda v: v.reshape(1, -1)
    full = lambda a: pl.BlockSpec(a.shape, lambda i: (0,) * a.ndim)

    weights = (W_p0, r1(b_p0), r1(g_p0), r1(be_p0),
               W_p1, r1(b_p1), r1(g_p1), r1(be_p1),
               W_in, r1(b_in), W_out, r1(b_out),
               W_e1, r1(b_e1),
               jnp.broadcast_to(W_e2, (8, W_e2.shape[1])),
               jnp.broadcast_to(b_e2.reshape(1, 1), (1, 8)))

    fused, ew = pl.pallas_call(
        _frontend_body,
        grid=(_B // _TRA,),
        in_specs=[pl.BlockSpec((_TRA, _D), lambda i: (i, 0)),
                  pl.BlockSpec((_TRA, _D), lambda i: (i, 0))]
                 + [full(w) for w in weights],
        out_specs=[pl.BlockSpec((_TRA, _H), lambda i: (i, 0)),
                   pl.BlockSpec((_TRA, 8), lambda i: (i, 0))],
        out_shape=[jax.ShapeDtypeStruct((_B, _H), jnp.float32),
                   jax.ShapeDtypeStruct((_B, 8), jnp.float32)],
    )(feat0, feat1, *weights)

    Hm = pl.pallas_call(
        _incidence_body,
        grid=(_B // _TCB,),
        in_specs=[pl.BlockSpec((_TCB, _H), lambda c: (c, 0)),
                  pl.BlockSpec((_B, _H), lambda c: (0, 0))],
        out_specs=pl.BlockSpec((_B, _TCB), lambda c: (0, c)),
        out_shape=jax.ShapeDtypeStruct((_B, _B), jnp.float32),
    )(fused, fused)

    return Hm, ew[:, 0]
